# baseline (speedup 1.0000x reference)
import functools

import jax
import jax.numpy as jnp
from jax import lax
from jax.experimental import pallas as pl
from jax.experimental.pallas import tpu as pltpu
from jax.experimental.pallas import tpu_sc as plsc

_D = 128
_B = 16384
_NC = 2
_NS = 16
_NW = _NC * _NS
_B_PER_W = _B // _NW
_CHUNK = 128
_N_CHUNKS = _B_PER_W // _CHUNK


def _sc_gather(table, idx3):
    mesh = plsc.VectorSubcoreMesh(core_axis_name="c", subcore_axis_name="s")

    @functools.partial(
        pl.kernel,
        mesh=mesh,
        out_type=jax.ShapeDtypeStruct((_B, _D), jnp.float32),
        scratch_types=[
            pltpu.VMEM((_N_CHUNKS, _CHUNK), jnp.int32),
            pltpu.VMEM((_B_PER_W, _D), jnp.float32),
            pltpu.SemaphoreType.DMA,
            pltpu.SemaphoreType.DMA,
        ],
    )
    def k(table_hbm, idx_hbm, out_hbm, idx_v, rows_v, gsem, wsem):
        wid = lax.axis_index("s") * _NC + lax.axis_index("c")
        base = wid * _B_PER_W
        pltpu.sync_copy(idx_hbm.at[wid], idx_v)
        copies = [
            pltpu.async_copy(
                table_hbm.at[idx_v.at[j]],
                rows_v.at[pl.ds(j * _CHUNK, _CHUNK)],
                gsem,
            )
            for j in range(_N_CHUNKS)
        ]
        for c in copies:
            c.wait()
        del wsem
        pltpu.sync_copy(rows_v, out_hbm.at[pl.ds(base, _B_PER_W)])

    return k(table, idx3)


_INV_DIV = None
_PHASE = None


def _inv_div():
    global _INV_DIV
    if _INV_DIV is None:
        k2 = (jnp.arange(_D, dtype=jnp.int32) // 2) * 2
        _INV_DIV = (10000.0 ** (-k2.astype(jnp.float32) / _D)).reshape(1, _D)
    return _INV_DIV


def _phase():
    global _PHASE
    if _PHASE is None:
        odd = (jnp.arange(_D, dtype=jnp.int32) & 1).astype(jnp.float32)
        _PHASE = (odd * jnp.float32(jnp.pi / 2)).reshape(1, _D)
    return _PHASE


_TC_ROWS = 1024


def _tc_sinusoid(tf):
    nb = tf.shape[0] // _TC_ROWS

    def body(t_ref, inv_ref, ph_ref, o_ref):
        o_ref[:, :] = jnp.sin(t_ref[:, :] * inv_ref[:, :] + ph_ref[:, :])

    return pl.pallas_call(
        body,
        grid=(nb,),
        in_specs=[
            pl.BlockSpec((_TC_ROWS, 1), lambda i: (i, 0)),
            pl.BlockSpec((1, _D), lambda i: (0, 0)),
            pl.BlockSpec((1, _D), lambda i: (0, 0)),
        ],
        out_specs=pl.BlockSpec((_TC_ROWS, _D), lambda i: (i, 0)),
        out_shape=jax.ShapeDtypeStruct((tf.shape[0], _D), jnp.float32),
    )(tf, _inv_div(), _phase())


@jax.jit
def kernel(t, pos_emb):
    tf = t.astype(jnp.float32).reshape(_B, 1)
    return _tc_sinusoid(tf)


# speedup vs baseline: 1.4252x; 1.4252x over previous
import functools

import jax
import jax.numpy as jnp
from jax import lax
from jax.experimental import pallas as pl
from jax.experimental.pallas import tpu as pltpu
from jax.experimental.pallas import tpu_sc as plsc

_D = 128
_B = 16384
_NC = 2
_NS = 16
_NW = _NC * _NS
_B_PER_W = _B // _NW
_CHUNK = 128
_N_CHUNKS = _B_PER_W // _CHUNK


def _sc_gather(table, idx3):
    mesh = plsc.VectorSubcoreMesh(core_axis_name="c", subcore_axis_name="s")

    @functools.partial(
        pl.kernel,
        mesh=mesh,
        out_type=jax.ShapeDtypeStruct((_B, _D), jnp.float32),
        scratch_types=[
            pltpu.VMEM((_N_CHUNKS, _CHUNK), jnp.int32),
            pltpu.VMEM((_B_PER_W, _D), jnp.float32),
            pltpu.SemaphoreType.DMA,
        ],
    )
    def k(table_hbm, idx_hbm, out_hbm, idx_v, rows_v, sem):
        wid = lax.axis_index("s") * _NC + lax.axis_index("c")
        pltpu.sync_copy(idx_hbm.at[wid], idx_v)
        copies = [
            pltpu.async_copy(
                table_hbm.at[idx_v.at[j]],
                rows_v.at[pl.ds(j * _CHUNK, _CHUNK)],
                sem,
            )
            for j in range(_N_CHUNKS)
        ]
        for c in copies:
            c.wait()
        pltpu.sync_copy(rows_v, out_hbm.at[pl.ds(wid * _B_PER_W, _B_PER_W)])

    return k(table, idx3)


@jax.jit
def kernel(t, pos_emb):
    idx3 = t.astype(jnp.int32).reshape(_NW, _N_CHUNKS, _CHUNK)
    return _sc_gather(pos_emb, idx3)
